# baseline (speedup 1.0000x reference)
import functools

import jax
import jax.numpy as jnp
from jax import lax
from jax.experimental import pallas as pl
from jax.experimental.pallas import tpu as pltpu
from jax.experimental.pallas import tpu_sc as plsc

B = 16384
S_ES = 64
D_REL = 128
NC = 2
NS = 16
L = 16
NW = NC * NS
BPW = B // NW
K = 32
NCHUNK = BPW // K

_S0 = 9.99996152e-01
_S1 = -1.66647032e-01
_S2 = 8.31724544e-03
_S3 = -1.93765902e-04
_S4 = 2.19812516e-06
_TWO_PI = 6.283185307179586
_INV_2PI = 0.15915494309189535
_RND = 12582912.0


def _sin(x):
    k = (x * _INV_2PI + _RND) - _RND
    xr = x - k * _TWO_PI
    s = xr * xr
    p = _S4
    p = p * s + _S3
    p = p * s + _S2
    p = p * s + _S1
    p = p * s + _S0
    return p * xr


def _body(s_h, o_h, r_h, d_h, h_h, *tabs_out_scratch):
    ent = tabs_out_scratch[0:14]
    R_f, R_i = tabs_out_scratch[14], tabs_out_scratch[15]
    out_h = tabs_out_scratch[16]
    sc = tabs_out_scratch[17:]
    si_v, oi_v, ri_v, d_v, h_v, out_v = sc[0:6]
    bufs_s = sc[6:20]
    bufs_o = sc[20:34]
    rf_v, ri_rel_v = sc[34], sc[35]
    sem = sc[36]

    wid = lax.axis_index("s") * NC + lax.axis_index("c")
    base = wid * BPW

    pltpu.sync_copy(s_h.at[pl.ds(base, BPW)], si_v)
    pltpu.sync_copy(o_h.at[pl.ds(base, BPW)], oi_v)
    pltpu.sync_copy(r_h.at[pl.ds(base, BPW)], ri_v)
    pltpu.sync_copy(d_h.at[pl.ds(base, BPW)], d_v)
    pltpu.sync_copy(h_h.at[pl.ds(base, BPW)], h_v)

    lane = lax.iota(jnp.int32, L)

    def chunk_body(c, carry):
        cbase = c * K
        cps = []
        for k in range(14):
            cps.append(pltpu.async_copy(
                ent[k].at[si_v.at[pl.ds(cbase, K)]], bufs_s[k], sem))
            cps.append(pltpu.async_copy(
                ent[k].at[oi_v.at[pl.ds(cbase, K)]], bufs_o[k], sem))
        cps.append(pltpu.async_copy(R_f.at[ri_v.at[pl.ds(cbase, K)]], rf_v, sem))
        cps.append(pltpu.async_copy(R_i.at[ri_v.at[pl.ds(cbase, K)]], ri_rel_v, sem))
        for cp in cps:
            cp.wait()

        def group_body(g, carry2):
            gbase = cbase + g * L

            def ex_body(l, svec):
                i = g * L + l
                idxv = lax.broadcast(gbase + l, (L,))
                db = plsc.load_gather(d_v, [idxv])
                hb = plsc.load_gather(h_v, [idxv])
                acc = jnp.zeros((L,), jnp.float32)
                for j in range(S_ES // L):
                    sl = pl.ds(j * L, L)
                    sh = pl.ds(S_ES + j * L, L)
                    es_s = bufs_s[0][i, sl]
                    eo_s = bufs_s[1][i, sl]
                    es_o = bufs_o[0][i, sl]
                    eo_o = bufs_o[1][i, sl]
                    ts_s = (bufs_s[10][i, sl] * _sin(db * bufs_s[2][i, sl] + bufs_s[6][i, sl])
                            + bufs_s[12][i, sl] * _sin(hb * bufs_s[4][i, sl] + bufs_s[8][i, sl]))
                    to_s = (bufs_s[11][i, sl] * _sin(db * bufs_s[3][i, sl] + bufs_s[7][i, sl])
                            + bufs_s[13][i, sl] * _sin(hb * bufs_s[5][i, sl] + bufs_s[9][i, sl]))
                    ts_o = (bufs_o[10][i, sl] * _sin(db * bufs_o[2][i, sl] + bufs_o[6][i, sl])
                            + bufs_o[12][i, sl] * _sin(hb * bufs_o[4][i, sl] + bufs_o[8][i, sl]))
                    to_o = (bufs_o[11][i, sl] * _sin(db * bufs_o[3][i, sl] + bufs_o[7][i, sl])
                            + bufs_o[13][i, sl] * _sin(hb * bufs_o[5][i, sl] + bufs_o[9][i, sl]))
                    acc = acc + es_s * rf_v[i, sl] * eo_o
                    acc = acc + ts_s * rf_v[i, sh] * to_o
                    acc = acc + es_o * ri_rel_v[i, sl] * eo_s
                    acc = acc + ts_o * ri_rel_v[i, sh] * to_s
                score = 0.5 * jnp.sum(acc)
                return jnp.where(lane == l, score, svec)

            svec = lax.fori_loop(0, L, ex_body, jnp.zeros((L,), jnp.float32))
            out_v[pl.ds(gbase, L)] = svec
            return carry2

        return lax.fori_loop(0, K // L, group_body, carry)

    lax.fori_loop(0, NCHUNK, chunk_body, 0)
    pltpu.sync_copy(out_v, out_h.at[pl.ds(base, BPW)])


@jax.jit
def _run(s, o, r, d, h, *tables):
    mesh = plsc.VectorSubcoreMesh(
        core_axis_name="c", subcore_axis_name="s", num_cores=NC, num_subcores=NS)
    scratch = (
        [pltpu.VMEM((BPW,), jnp.int32)] * 3
        + [pltpu.VMEM((BPW,), jnp.float32)] * 2
        + [pltpu.VMEM((BPW,), jnp.float32)]
        + [pltpu.VMEM((K, S_ES), jnp.float32)] * 28
        + [pltpu.VMEM((K, D_REL), jnp.float32)] * 2
        + [pltpu.SemaphoreType.DMA]
    )
    fn = pl.kernel(
        _body,
        out_type=jax.ShapeDtypeStruct((B,), jnp.float32),
        mesh=mesh,
        scratch_types=scratch,
        compiler_params=pltpu.CompilerParams(
            needs_layout_passes=False, use_tc_tiling_on_sc=False),
    )
    return fn(s, o, r, d, h, *tables)


def kernel(s, o, r, t, E_s, E_o, R_f, R_i,
           d_frq_s, d_frq_o, h_frq_s, h_frq_o,
           d_phi_s, d_phi_o, h_phi_s, h_phi_o,
           d_amp_s, d_amp_o, h_amp_s, h_amp_o):
    d = t[:, 0].astype(jnp.float32)
    h = t[:, 1].astype(jnp.float32)
    ent = (E_s, E_o,
           d_frq_s, d_frq_o, h_frq_s, h_frq_o,
           d_phi_s, d_phi_o, h_phi_s, h_phi_o,
           d_amp_s, d_amp_o, h_amp_s, h_amp_o)
    return _run(s, o, r, d, h, *ent, R_f, R_i)


# speedup vs baseline: 1.1539x; 1.1539x over previous
import functools

import jax
import jax.numpy as jnp
from jax import lax
from jax.experimental import pallas as pl
from jax.experimental.pallas import tpu as pltpu
from jax.experimental.pallas import tpu_sc as plsc

B = 16384
S_ES = 64
DPAIR = 128
NPAIR = 7
NC = 2
NS = 16
L = 16
NW = NC * NS
BPW = B // NW
K = 32
NCHUNK = BPW // K

_S0 = 9.99996152e-01
_S1 = -1.66647032e-01
_S2 = 8.31724544e-03
_S3 = -1.93765902e-04
_S4 = 2.19812516e-06
_TWO_PI = 6.283185307179586
_INV_2PI = 0.15915494309189535
_RND = 12582912.0


def _sin(x):
    k = (x * _INV_2PI + _RND) - _RND
    xr = x - k * _TWO_PI
    s = xr * xr
    p = _S4
    p = p * s + _S3
    p = p * s + _S2
    p = p * s + _S1
    p = p * s + _S0
    return p * xr


def _body(s_h, o_h, r_h, d_h, h_h, *rest):
    pair_h = rest[0:NPAIR]
    rf_h, ri_h = rest[NPAIR], rest[NPAIR + 1]
    out_h = rest[NPAIR + 2]
    sc = rest[NPAIR + 3:]
    si_v, oi_v, ri_v, d_v, h_v, out_v = sc[0:6]
    bufS = sc[6:6 + NPAIR]
    bufO = sc[6 + NPAIR:6 + 2 * NPAIR]
    bufRf, bufRi = sc[6 + 2 * NPAIR], sc[7 + 2 * NPAIR]
    sem = sc[8 + 2 * NPAIR]

    wid = lax.axis_index("s") * NC + lax.axis_index("c")
    base = wid * BPW

    pltpu.sync_copy(s_h.at[pl.ds(base, BPW)], si_v)
    pltpu.sync_copy(o_h.at[pl.ds(base, BPW)], oi_v)
    pltpu.sync_copy(r_h.at[pl.ds(base, BPW)], ri_v)
    pltpu.sync_copy(d_h.at[pl.ds(base, BPW)], d_v)
    pltpu.sync_copy(h_h.at[pl.ds(base, BPW)], h_v)

    lane = lax.iota(jnp.int32, L)


    def chunk_body(c, carry):
        cbase = c * K
        cps = []
        for p in range(NPAIR):
            cps.append(pltpu.async_copy(
                pair_h[p].at[si_v.at[pl.ds(cbase, K)]], bufS[p], sem))
            cps.append(pltpu.async_copy(
                pair_h[p].at[oi_v.at[pl.ds(cbase, K)]], bufO[p], sem))
        cps.append(pltpu.async_copy(rf_h.at[ri_v.at[pl.ds(cbase, K)]], bufRf, sem))
        cps.append(pltpu.async_copy(ri_h.at[ri_v.at[pl.ds(cbase, K)]], bufRi, sem))
        for cp in cps:
            cp.wait()

        def group_body(g, carry2):
            gbase = cbase + g * L

            def ex_body(l, svec):
                i = g * L + l
                idxv = lax.broadcast(gbase + l, (L,))
                db = plsc.load_gather(d_v, [idxv])
                hb = plsc.load_gather(h_v, [idxv])
                acc = jnp.zeros((L,), jnp.float32)
                for j in range(S_ES // L):
                    def S(k):
                        return bufS[k // 2][i, pl.ds((k % 2) * S_ES + j * L, L)]
                    def O(k):
                        return bufO[k // 2][i, pl.ds((k % 2) * S_ES + j * L, L)]
                    ts_s = (S(10) * _sin(db * S(2) + S(6))
                            + S(12) * _sin(hb * S(4) + S(8)))
                    to_s = (S(11) * _sin(db * S(3) + S(7))
                            + S(13) * _sin(hb * S(5) + S(9)))
                    ts_o = (O(10) * _sin(db * O(2) + O(6))
                            + O(12) * _sin(hb * O(4) + O(8)))
                    to_o = (O(11) * _sin(db * O(3) + O(7))
                            + O(13) * _sin(hb * O(5) + O(9)))
                    rf_lo = bufRf[i, pl.ds(j * L, L)]
                    rf_hi = bufRf[i, pl.ds(S_ES + j * L, L)]
                    ri_lo = bufRi[i, pl.ds(j * L, L)]
                    ri_hi = bufRi[i, pl.ds(S_ES + j * L, L)]
                    acc = acc + S(0) * rf_lo * O(1)
                    acc = acc + ts_s * rf_hi * to_o
                    acc = acc + O(0) * ri_lo * S(1)
                    acc = acc + ts_o * ri_hi * to_s
                score = 0.5 * jnp.sum(acc)
                return jnp.where(lane == l, score, svec)

            svec = lax.fori_loop(0, L, ex_body, jnp.zeros((L,), jnp.float32))
            out_v[pl.ds(gbase, L)] = svec
            return carry2

        return lax.fori_loop(0, K // L, group_body, carry)

    lax.fori_loop(0, NCHUNK, chunk_body, 0)
    pltpu.sync_copy(out_v, out_h.at[pl.ds(base, BPW)])


@jax.jit
def _run(s, o, r, d, h, *tables):
    mesh = plsc.VectorSubcoreMesh(
        core_axis_name="c", subcore_axis_name="s", num_cores=NC, num_subcores=NS)
    scratch = (
        [pltpu.VMEM((BPW,), jnp.int32)] * 3
        + [pltpu.VMEM((BPW,), jnp.float32)] * 2
        + [pltpu.VMEM((BPW,), jnp.float32)]
        + [pltpu.VMEM((K, DPAIR), jnp.float32)] * (2 * NPAIR + 2)
        + [pltpu.SemaphoreType.DMA]
    )
    fn = pl.kernel(
        _body,
        out_type=jax.ShapeDtypeStruct((B,), jnp.float32),
        mesh=mesh,
        scratch_types=scratch,
        compiler_params=pltpu.CompilerParams(
            needs_layout_passes=False, use_tc_tiling_on_sc=True),
    )
    return fn(s, o, r, d, h, *tables)


def kernel(s, o, r, t, E_s, E_o, R_f, R_i,
           d_frq_s, d_frq_o, h_frq_s, h_frq_o,
           d_phi_s, d_phi_o, h_phi_s, h_phi_o,
           d_amp_s, d_amp_o, h_amp_s, h_amp_o):
    d = t[:, 0].astype(jnp.float32)
    h = t[:, 1].astype(jnp.float32)
    ent = (E_s, E_o,
           d_frq_s, d_frq_o, h_frq_s, h_frq_o,
           d_phi_s, d_phi_o, h_phi_s, h_phi_o,
           d_amp_s, d_amp_o, h_amp_s, h_amp_o)
    pairs = tuple(jnp.concatenate([ent[2 * p], ent[2 * p + 1]], axis=1)
                  for p in range(NPAIR))
    return _run(s, o, r, d, h, *pairs, R_f, R_i)


# speedup vs baseline: 1.2747x; 1.1047x over previous
import functools

import jax
import jax.numpy as jnp
from jax import lax
from jax.experimental import pallas as pl
from jax.experimental.pallas import tpu as pltpu
from jax.experimental.pallas import tpu_sc as plsc

B = 16384
S_ES = 64
DPAIR = 128
NPAIR = 7
NC = 2
NS = 16
L = 16
NW = NC * NS
BPW = B // NW
K = 16
NCHUNK = BPW // K

_S0 = 9.99996152e-01
_S1 = -1.66647032e-01
_S2 = 8.31724544e-03
_S3 = -1.93765902e-04
_S4 = 2.19812516e-06
_TWO_PI = 6.283185307179586
_INV_2PI = 0.15915494309189535
_RND = 12582912.0


def _sin(x):
    k = (x * _INV_2PI + _RND) - _RND
    xr = x - k * _TWO_PI
    s = xr * xr
    p = _S4
    p = p * s + _S3
    p = p * s + _S2
    p = p * s + _S1
    p = p * s + _S0
    return p * xr


def _body(s_h, o_h, r_h, d_h, h_h, *rest):
    pair_h = rest[0:NPAIR]
    rf_h, ri_h = rest[NPAIR], rest[NPAIR + 1]
    out_h = rest[NPAIR + 2]
    sc = rest[NPAIR + 3:]
    si_v, oi_v, ri_v, d_v, h_v, out_v = sc[0:6]
    nbuf = 2 * NPAIR + 2
    sets = [sc[6 + t * nbuf:6 + (t + 1) * nbuf] for t in range(2)]
    sems = sc[6 + 2 * nbuf:6 + 2 * nbuf + 2]

    wid = lax.axis_index("s") * NC + lax.axis_index("c")
    base = wid * BPW

    pltpu.sync_copy(s_h.at[pl.ds(base, BPW)], si_v)
    pltpu.sync_copy(o_h.at[pl.ds(base, BPW)], oi_v)
    pltpu.sync_copy(r_h.at[pl.ds(base, BPW)], ri_v)
    pltpu.sync_copy(d_h.at[pl.ds(base, BPW)], d_v)
    pltpu.sync_copy(h_h.at[pl.ds(base, BPW)], h_v)

    lane = lax.iota(jnp.int32, L)

    def _descs(cbase, t):
        bufs = sets[t]
        sem = sems[t]
        d = []
        for p in range(NPAIR):
            d.append((pair_h[p].at[si_v.at[pl.ds(cbase, K)]], bufs[p], sem))
            d.append((pair_h[p].at[oi_v.at[pl.ds(cbase, K)]], bufs[NPAIR + p], sem))
        d.append((rf_h.at[ri_v.at[pl.ds(cbase, K)]], bufs[2 * NPAIR], sem))
        d.append((ri_h.at[ri_v.at[pl.ds(cbase, K)]], bufs[2 * NPAIR + 1], sem))
        return d

    def fire(cbase, t):
        for src, dst, sem in _descs(cbase, t):
            pltpu.async_copy(src, dst, sem)

    def drain(cbase, t):
        for src, dst, sem in _descs(cbase, t):
            pltpu.make_async_copy(src, dst, sem).wait()


    def compute(cbase, t):
        bufs = sets[t]
        bufS = bufs[0:NPAIR]
        bufO = bufs[NPAIR:2 * NPAIR]
        bufRf, bufRi = bufs[2 * NPAIR], bufs[2 * NPAIR + 1]

        def ex_body(l, svec):
            idxv = lax.broadcast(cbase + l, (L,))
            db = plsc.load_gather(d_v, [idxv])
            hb = plsc.load_gather(h_v, [idxv])
            acc = jnp.zeros((L,), jnp.float32)
            for j in range(S_ES // L):
                def S(k):
                    return bufS[k // 2][l, pl.ds((k % 2) * S_ES + j * L, L)]
                def O(k):
                    return bufO[k // 2][l, pl.ds((k % 2) * S_ES + j * L, L)]
                ts_s = (S(10) * _sin(db * S(2) + S(6))
                        + S(12) * _sin(hb * S(4) + S(8)))
                to_s = (S(11) * _sin(db * S(3) + S(7))
                        + S(13) * _sin(hb * S(5) + S(9)))
                ts_o = (O(10) * _sin(db * O(2) + O(6))
                        + O(12) * _sin(hb * O(4) + O(8)))
                to_o = (O(11) * _sin(db * O(3) + O(7))
                        + O(13) * _sin(hb * O(5) + O(9)))
                rf_lo = bufRf[l, pl.ds(j * L, L)]
                rf_hi = bufRf[l, pl.ds(S_ES + j * L, L)]
                ri_lo = bufRi[l, pl.ds(j * L, L)]
                ri_hi = bufRi[l, pl.ds(S_ES + j * L, L)]
                acc = acc + S(0) * rf_lo * O(1)
                acc = acc + ts_s * rf_hi * to_o
                acc = acc + O(0) * ri_lo * S(1)
                acc = acc + ts_o * ri_hi * to_s
            score = 0.5 * jnp.sum(acc)
            return jnp.where(lane == l, score, svec)

        svec = lax.fori_loop(0, L, ex_body, jnp.zeros((L,), jnp.float32))
        out_v[pl.ds(cbase, L)] = svec

    fire(0, 0)

    def loop_body(g, carry):
        c0 = 2 * g
        c1 = 2 * g + 1
        fire(c1 * K, 1)
        drain(c0 * K, 0)
        compute(c0 * K, 0)
        nxt0 = jnp.minimum(c0 + 2, NCHUNK - 1) * K
        fire(nxt0, 0)
        drain(c1 * K, 1)
        compute(c1 * K, 1)
        return carry

    lax.fori_loop(0, NCHUNK // 2, loop_body, 0)
    drain((NCHUNK - 1) * K, 0)

    pltpu.sync_copy(out_v, out_h.at[pl.ds(base, BPW)])


@jax.jit
def _run(s, o, r, d, h, *tables):
    mesh = plsc.VectorSubcoreMesh(
        core_axis_name="c", subcore_axis_name="s", num_cores=NC, num_subcores=NS)
    scratch = (
        [pltpu.VMEM((BPW,), jnp.int32)] * 3
        + [pltpu.VMEM((BPW,), jnp.float32)] * 2
        + [pltpu.VMEM((BPW,), jnp.float32)]
        + [pltpu.VMEM((K, DPAIR), jnp.float32)] * (2 * (2 * NPAIR + 2))
        + [pltpu.SemaphoreType.DMA] * 2
    )
    fn = pl.kernel(
        _body,
        out_type=jax.ShapeDtypeStruct((B,), jnp.float32),
        mesh=mesh,
        scratch_types=scratch,
        compiler_params=pltpu.CompilerParams(
            needs_layout_passes=False, use_tc_tiling_on_sc=True),
    )
    return fn(s, o, r, d, h, *tables)


def kernel(s, o, r, t, E_s, E_o, R_f, R_i,
           d_frq_s, d_frq_o, h_frq_s, h_frq_o,
           d_phi_s, d_phi_o, h_phi_s, h_phi_o,
           d_amp_s, d_amp_o, h_amp_s, h_amp_o):
    d = t[:, 0].astype(jnp.float32)
    h = t[:, 1].astype(jnp.float32)
    ent = (E_s, E_o,
           d_frq_s, d_frq_o, h_frq_s, h_frq_o,
           d_phi_s, d_phi_o, h_phi_s, h_phi_o,
           d_amp_s, d_amp_o, h_amp_s, h_amp_o)
    pairs = tuple(jnp.concatenate([ent[2 * p], ent[2 * p + 1]], axis=1)
                  for p in range(NPAIR))
    return _run(s, o, r, d, h, *pairs, R_f, R_i)


# speedup vs baseline: 2.0715x; 1.6251x over previous
import functools

import jax
import jax.numpy as jnp
from jax import lax
from jax.experimental import pallas as pl
from jax.experimental.pallas import tpu as pltpu
from jax.experimental.pallas import tpu_sc as plsc

B = 16384
S_ES = 64
DPAIR = 128
NPAIR = 7
NC = 2
NS = 16
L = 16
NW = NC * NS
BPW = B // NW
K = 16
NCHUNK = BPW // K

_S0 = 9.99996152e-01
_S1 = -1.66647032e-01
_S2 = 8.31724544e-03
_S3 = -1.93765902e-04
_S4 = 2.19812516e-06
_TWO_PI = 6.283185307179586
_INV_2PI = 0.15915494309189535
_RND = 12582912.0


def _sin(x):
    k = (x * _INV_2PI + _RND) - _RND
    xr = x - k * _TWO_PI
    s = xr * xr
    p = _S4
    p = p * s + _S3
    p = p * s + _S2
    p = p * s + _S1
    p = p * s + _S0
    return p * xr


def _body(s_h, o_h, r_h, d_h, h_h, *rest):
    pair_h = rest[0:NPAIR]
    rf_h, ri_h = rest[NPAIR], rest[NPAIR + 1]
    out_h = rest[NPAIR + 2]
    sc = rest[NPAIR + 3:]
    si_v, oi_v, ri_v, d_v, h_v, out_v = sc[0:6]
    nbuf = 2 * NPAIR + 2
    sets = [sc[6 + t * nbuf:6 + (t + 1) * nbuf] for t in range(2)]
    sems = sc[6 + 2 * nbuf:6 + 2 * nbuf + 2]

    wid = lax.axis_index("s") * NC + lax.axis_index("c")
    base = wid * BPW

    pltpu.sync_copy(s_h.at[pl.ds(base, BPW)], si_v)
    pltpu.sync_copy(o_h.at[pl.ds(base, BPW)], oi_v)
    pltpu.sync_copy(r_h.at[pl.ds(base, BPW)], ri_v)
    pltpu.sync_copy(d_h.at[pl.ds(base, BPW)], d_v)
    pltpu.sync_copy(h_h.at[pl.ds(base, BPW)], h_v)

    lane = lax.iota(jnp.int32, L)

    def _descs(cbase, t):
        bufs = sets[t]
        sem = sems[t]
        d = []
        for p in range(NPAIR):
            d.append((pair_h[p].at[si_v.at[pl.ds(cbase, K)]], bufs[p], sem))
            d.append((pair_h[p].at[oi_v.at[pl.ds(cbase, K)]], bufs[NPAIR + p], sem))
        d.append((rf_h.at[ri_v.at[pl.ds(cbase, K)]], bufs[2 * NPAIR], sem))
        d.append((ri_h.at[ri_v.at[pl.ds(cbase, K)]], bufs[2 * NPAIR + 1], sem))
        return d

    def fire(cbase, t):
        for src, dst, sem in _descs(cbase, t):
            pltpu.async_copy(src, dst, sem)

    def drain(cbase, t):
        for src, dst, sem in _descs(cbase, t):
            pltpu.make_async_copy(src, dst, sem).wait()


    def compute(cbase, t):
        bufs = sets[t]
        bufS = bufs[0:NPAIR]
        bufO = bufs[NPAIR:2 * NPAIR]
        bufRf, bufRi = bufs[2 * NPAIR], bufs[2 * NPAIR + 1]

        def ex_body(l, svec):
            idxv = lax.broadcast(cbase + l, (L,))
            db = plsc.load_gather(d_v, [idxv])
            hb = plsc.load_gather(h_v, [idxv])
            acc = jnp.zeros((L,), jnp.float32)
            for j in range(S_ES // L):
                def S(k):
                    return bufS[k // 2][l, pl.ds((k % 2) * S_ES + j * L, L)]
                def O(k):
                    return bufO[k // 2][l, pl.ds((k % 2) * S_ES + j * L, L)]
                ts_s = (S(10) * _sin(db * S(2) + S(6))
                        + S(12) * _sin(hb * S(4) + S(8)))
                to_s = (S(11) * _sin(db * S(3) + S(7))
                        + S(13) * _sin(hb * S(5) + S(9)))
                ts_o = (O(10) * _sin(db * O(2) + O(6))
                        + O(12) * _sin(hb * O(4) + O(8)))
                to_o = (O(11) * _sin(db * O(3) + O(7))
                        + O(13) * _sin(hb * O(5) + O(9)))
                rf_lo = bufRf[l, pl.ds(j * L, L)]
                rf_hi = bufRf[l, pl.ds(S_ES + j * L, L)]
                ri_lo = bufRi[l, pl.ds(j * L, L)]
                ri_hi = bufRi[l, pl.ds(S_ES + j * L, L)]
                acc = acc + S(0) * rf_lo * O(1)
                acc = acc + ts_s * rf_hi * to_o
                acc = acc + O(0) * ri_lo * S(1)
                acc = acc + ts_o * ri_hi * to_s
            score = 0.5 * jnp.sum(acc)
            return jnp.where(lane == l, score, svec)

        svec = lax.fori_loop(0, L, ex_body, jnp.zeros((L,), jnp.float32))
        out_v[pl.ds(cbase, L)] = svec

    fire(0, 0)

    def loop_body(g, carry):
        c0 = 2 * g
        c1 = 2 * g + 1
        fire(c1 * K, 1)
        drain(c0 * K, 0)
        compute(c0 * K, 0)
        nxt0 = jnp.minimum(c0 + 2, NCHUNK - 1) * K
        fire(nxt0, 0)
        drain(c1 * K, 1)
        compute(c1 * K, 1)
        return carry

    lax.fori_loop(0, NCHUNK // 2, loop_body, 0)
    drain((NCHUNK - 1) * K, 0)

    pltpu.sync_copy(out_v, out_h.at[pl.ds(base, BPW)])


_EB = 1024
_NEB = (100000 + _EB - 1) // _EB


def _tr_body(*refs):
    ins = refs[:14]
    outs = refs[14:]
    for p in range(NPAIR):
        a = ins[2 * p][...]
        b = ins[2 * p + 1][...]
        outs[p][:, 0:S_ES] = a.T
        outs[p][:, S_ES:DPAIR] = b.T


def _transpose_pairs(*tabs_t):
    grid = (_NEB,)
    in_specs = [pl.BlockSpec((S_ES, _EB), lambda i: (0, i)) for _ in range(14)]
    out_specs = [pl.BlockSpec((_EB, DPAIR), lambda i: (i, 0)) for _ in range(NPAIR)]
    out_shape = [jax.ShapeDtypeStruct((100000, DPAIR), jnp.float32)
                 for _ in range(NPAIR)]
    return pl.pallas_call(
        _tr_body, grid=grid, in_specs=in_specs, out_specs=out_specs,
        out_shape=out_shape)(*tabs_t)


def _run(s, o, r, d, h, *tables):
    mesh = plsc.VectorSubcoreMesh(
        core_axis_name="c", subcore_axis_name="s", num_cores=NC, num_subcores=NS)
    scratch = (
        [pltpu.VMEM((BPW,), jnp.int32)] * 3
        + [pltpu.VMEM((BPW,), jnp.float32)] * 2
        + [pltpu.VMEM((BPW,), jnp.float32)]
        + [pltpu.VMEM((K, DPAIR), jnp.float32)] * (2 * (2 * NPAIR + 2))
        + [pltpu.SemaphoreType.DMA] * 2
    )
    fn = pl.kernel(
        _body,
        out_type=jax.ShapeDtypeStruct((B,), jnp.float32),
        mesh=mesh,
        scratch_types=scratch,
        compiler_params=pltpu.CompilerParams(
            needs_layout_passes=False, use_tc_tiling_on_sc=True),
    )
    return fn(s, o, r, d, h, *tables)


@jax.jit
def kernel(s, o, r, t, E_s, E_o, R_f, R_i,
           d_frq_s, d_frq_o, h_frq_s, h_frq_o,
           d_phi_s, d_phi_o, h_phi_s, h_phi_o,
           d_amp_s, d_amp_o, h_amp_s, h_amp_o):
    d = t[:, 0].astype(jnp.float32)
    h = t[:, 1].astype(jnp.float32)
    ent = (E_s, E_o,
           d_frq_s, d_frq_o, h_frq_s, h_frq_o,
           d_phi_s, d_phi_o, h_phi_s, h_phi_o,
           d_amp_s, d_amp_o, h_amp_s, h_amp_o)
    pairs = _transpose_pairs(*(e.T for e in ent))
    return _run(s, o, r, d, h, *pairs, R_f, R_i)
